# baseline (speedup 1.0000x reference)
import jax
import jax.numpy as jnp
from jax import lax
from jax.experimental import pallas as pl
from jax.experimental.pallas import tpu as pltpu
from jax.experimental.pallas import tpu_sc as plsc

N = 10000
E = 320000
D = 128
NCLASS = 64

NC = 2
NS = 16
DH = D // NC
CHUNK = 128
NCH = E // CHUNK
ROWS_PT = N // NS
HW = 16

GC_PT = NCH // NS
GC_EXTRA = NCH - GC_PT * NS
GC_NBUF = 4
DG_SC = NCH // NC
DG_PT = DG_SC // NS
DG_EXTRA = DG_SC - DG_PT * NS


def _sc_mesh():
    return plsc.VectorSubcoreMesh(
        core_axis_name="c", subcore_axis_name="s", num_cores=NC,
        num_subcores=NS,
    )


_SC_PARAMS = pltpu.CompilerParams(use_tc_tiling_on_sc=False)


def _deg_body(edge_hbm, hist_hbm, src_buf, dst_buf, obuf, zbuf, cb,
              hs_src, hs_dst, sem_s, sem_d):
    c = lax.axis_index("c")
    s = lax.axis_index("s")
    src_hbm = edge_hbm.at[0]
    dst_hbm = edge_hbm.at[1]
    base = c * DG_SC + s * DG_PT

    pltpu.sync_copy(src_hbm.at[pl.ds(base, DG_PT)],
                    src_buf.at[pl.ds(0, DG_PT)])
    pltpu.sync_copy(dst_hbm.at[pl.ds(base, DG_PT)],
                    dst_buf.at[pl.ds(0, DG_PT)])

    @pl.when(s < DG_EXTRA)
    def _():
        extra = c * DG_SC + NS * DG_PT + s
        pltpu.sync_copy(src_hbm.at[extra], src_buf.at[DG_PT])
        pltpu.sync_copy(dst_hbm.at[extra], dst_buf.at[DG_PT])

    lane = lax.iota(jnp.int32, 16)
    one_line = jnp.ones((16,), jnp.float32)
    zline = jnp.zeros((16,), jnp.float32)

    def fill(i, _):
        obuf[i, :] = one_line
        return 0
    lax.fori_loop(0, CHUNK, fill, 0)

    def zfill(i, _):
        zbuf[i, :] = zline
        return 0
    lax.fori_loop(0, ROWS_PT, zfill, 0)

    pltpu.sync_copy(zbuf, hs_src.at[pl.ds(s * ROWS_PT, ROWS_PT)])
    pltpu.sync_copy(zbuf, hs_dst.at[pl.ds(s * ROWS_PT, ROWS_PT)])
    plsc.subcore_barrier()

    nmine = DG_PT + jnp.where(
        s < DG_EXTRA, jnp.int32(1), jnp.int32(0))

    def step(j, _):
        pltpu.async_copy(obuf, hs_src.at[src_buf.at[j]], sem_s, add=True)
        pltpu.async_copy(obuf, hs_dst.at[dst_buf.at[j]], sem_d, add=True)
        return 0
    lax.fori_loop(0, nmine, step, 0)

    def drain(j, _):
        pltpu.make_async_copy(obuf, hs_src.at[src_buf.at[0]], sem_s).wait()
        pltpu.make_async_copy(obuf, hs_dst.at[dst_buf.at[0]], sem_d).wait()
        return 0
    lax.fori_loop(0, nmine, drain, 0)

    plsc.subcore_barrier()

    rows = pl.ds(s * ROWS_PT, ROWS_PT)
    for h, hs in ((0, hs_src), (1, hs_dst)):
        pltpu.sync_copy(hs.at[rows], zbuf)
        for i in range(ROWS_PT // 16 + 1):
            r0 = min(i * 16, ROWS_PT - 16)
            w = zbuf[r0, :]
            for k in range(1, 16):
                w = jnp.where(lane == k, zbuf[r0 + k, :], w)
            cb[pl.ds(r0, 16)] = w
        pltpu.sync_copy(cb, hist_hbm.at[c, h, s])


def _degrees(edge3):
    f = pl.kernel(
        _deg_body,
        out_type=jax.ShapeDtypeStruct((NC, 2, NS, 640), jnp.float32),
        mesh=_sc_mesh(),
        scratch_types=[
            pltpu.VMEM((DG_PT + 1, CHUNK), jnp.int32),
            pltpu.VMEM((DG_PT + 1, CHUNK), jnp.int32),
            pltpu.VMEM((CHUNK, HW), jnp.float32),
            pltpu.VMEM((ROWS_PT, HW), jnp.float32),
            pltpu.VMEM((640,), jnp.float32),
            pltpu.VMEM_SHARED((N, HW), jnp.float32),
            pltpu.VMEM_SHARED((N, HW), jnp.float32),
            pltpu.SemaphoreType.DMA,
            pltpu.SemaphoreType.DMA,
        ],
        compiler_params=_SC_PARAMS,
    )
    return f(edge3)


def _gconv_body(m_hbm, edge_hbm, out_hbm, src_buf, dst_buf,
                rb0, rb1, rb2, rb3, zbuf, agg_sh, gsem, ssem):
    c = lax.axis_index("c")
    s = lax.axis_index("s")
    src_hbm = edge_hbm.at[0]
    dst_hbm = edge_hbm.at[1]
    rb = (rb0, rb1, rb2, rb3)
    mh = m_hbm.at[c]

    pltpu.sync_copy(src_hbm.at[pl.ds(s * GC_PT, GC_PT)], src_buf)
    pltpu.sync_copy(dst_hbm.at[pl.ds(s * GC_PT, GC_PT)], dst_buf)

    zline = jnp.zeros((16,), jnp.float32)

    def zfill(i, _):
        for k in range(DH // 16):
            zbuf[i, pl.ds(k * 16, 16)] = zline
        return 0
    lax.fori_loop(0, ROWS_PT // 5, zfill, 0)

    for t in range(5):
        pltpu.sync_copy(
            zbuf, agg_sh.at[pl.ds(s * ROWS_PT + t * (ROWS_PT // 5),
                                  ROWS_PT // 5)])
    plsc.subcore_barrier()

    def start_gather(j, b):
        pltpu.async_copy(mh.at[src_buf.at[j]], rb[b], gsem.at[b])

    def wait_gather(b):
        pltpu.make_async_copy(mh.at[src_buf.at[0]], rb[b], gsem.at[b]).wait()

    def start_scatter(j, b):
        pltpu.async_copy(rb[b], agg_sh.at[dst_buf.at[j]], ssem.at[b],
                         add=True)

    def wait_scatter(b):
        pltpu.make_async_copy(
            rb[b], agg_sh.at[dst_buf.at[0]], ssem.at[b]).wait()

    for b in range(GC_NBUF):
        start_gather(b, b)

    def step(g, _):
        j0 = g * GC_NBUF
        for b in range(GC_NBUF):
            wait_gather(b)
            start_scatter(j0 + b, b)
        for b in range(GC_NBUF):
            wait_scatter(b)
            start_gather(j0 + GC_NBUF + b, b)
        return 0
    lax.fori_loop(0, GC_PT // GC_NBUF - 1, step, 0)

    j0 = GC_PT - GC_NBUF
    for b in range(GC_NBUF):
        wait_gather(b)
        start_scatter(j0 + b, b)
    for b in range(GC_NBUF):
        wait_scatter(b)

    @pl.when(s < GC_EXTRA)
    def _():
        pltpu.sync_copy(src_hbm.at[NS * GC_PT + s], src_buf.at[0])
        pltpu.sync_copy(dst_hbm.at[NS * GC_PT + s], dst_buf.at[0])
        pltpu.async_copy(mh.at[src_buf.at[0]], rb[0], gsem.at[0])
        wait_gather(0)
        pltpu.sync_copy(rb[0], agg_sh.at[dst_buf.at[0]], add=True)

    plsc.subcore_barrier()
    rows = pl.ds(s * ROWS_PT, ROWS_PT)
    pltpu.sync_copy(agg_sh.at[rows], out_hbm.at[rows, pl.ds(c * DH, DH)])


def _gconv(m, edge3):
    f = pl.kernel(
        _gconv_body,
        out_type=jax.ShapeDtypeStruct((N, D), jnp.float32),
        mesh=_sc_mesh(),
        scratch_types=[
            pltpu.VMEM((GC_PT, CHUNK), jnp.int32),
            pltpu.VMEM((GC_PT, CHUNK), jnp.int32),
        ] + [pltpu.VMEM((CHUNK, DH), jnp.float32) for _ in range(GC_NBUF)] + [
            pltpu.VMEM((ROWS_PT // 5, DH), jnp.float32),
            pltpu.VMEM_SHARED((N, DH), jnp.float32),
            pltpu.SemaphoreType.DMA((GC_NBUF,)),
            pltpu.SemaphoreType.DMA((GC_NBUF,)),
        ],
        compiler_params=_SC_PARAMS,
    )
    return f(m, edge3)


BLK = 1000


def _dot3(a, b):
    ah = a.astype(jnp.bfloat16)
    al = (a - ah.astype(jnp.float32)).astype(jnp.bfloat16)
    bh = b.astype(jnp.bfloat16)
    bl = (b - bh.astype(jnp.float32)).astype(jnp.bfloat16)
    f = lambda p, q: jax.lax.dot_general(
        p, q, (((1,), (0,)), ((), ())),
        preferred_element_type=jnp.float32)
    return f(ah, bh) + f(ah, bl) + f(al, bh)


def _tc0_body(x_ref, w1_ref, b1_ref, w2_ref, b2_ref, h_ref):
    h = _dot3(x_ref[...], w1_ref[...]) + b1_ref[...]
    h_ref[...] = _dot3(h, w2_ref[...]) + b2_ref[...]


def _tc0(x, W1, b1, W2, b2):
    full = lambda *shape: pl.BlockSpec(shape, lambda i: (0,) * len(shape))
    return pl.pallas_call(
        _tc0_body,
        grid=(N // BLK,),
        in_specs=[
            pl.BlockSpec((BLK, D), lambda i: (i, 0)),
            full(D, D), full(D), full(D, D), full(D),
        ],
        out_specs=pl.BlockSpec((BLK, D), lambda i: (i, 0)),
        out_shape=jax.ShapeDtypeStruct((N, D), jnp.float32),
    )(x, W1, b1, W2, b2)


def _tc1_body(h_ref, wg0_ref, nrm_ref, m1_ref):
    z = _dot3(h_ref[...], wg0_ref[...])
    m = z * nrm_ref[:, 0][:, None]
    m1_ref[0] = m[:, :DH]
    m1_ref[1] = m[:, DH:]


def _tc1(h, Wg0, nrm):
    full = lambda *shape: pl.BlockSpec(shape, lambda i: (0,) * len(shape))
    return pl.pallas_call(
        _tc1_body,
        grid=(N // BLK,),
        in_specs=[
            pl.BlockSpec((BLK, D), lambda i: (i, 0)),
            full(D, D),
            pl.BlockSpec((BLK, 8), lambda i: (i, 0)),
        ],
        out_specs=pl.BlockSpec((NC, BLK, DH), lambda i: (0, i, 0)),
        out_shape=jax.ShapeDtypeStruct((NC, N, DH), jnp.float32),
    )(h, Wg0, nrm)


def _tc2_body(agg_ref, nrm_ref, bg0_ref, wg1_ref, m2_ref):
    h1 = jnp.maximum(
        agg_ref[...] * nrm_ref[:, 1][:, None] + bg0_ref[...], 0.0)
    m = _dot3(h1, wg1_ref[...]) * nrm_ref[:, 0][:, None]
    m2_ref[0] = m[:, :DH]
    m2_ref[1] = m[:, DH:]


def _tc2(agg1, nrm, bg0, Wg1):
    full = lambda *shape: pl.BlockSpec(shape, lambda i: (0,) * len(shape))
    return pl.pallas_call(
        _tc2_body,
        grid=(N // BLK,),
        in_specs=[
            pl.BlockSpec((BLK, D), lambda i: (i, 0)),
            pl.BlockSpec((BLK, 8), lambda i: (i, 0)),
            full(D), full(D, D),
        ],
        out_specs=pl.BlockSpec((NC, BLK, DH), lambda i: (0, i, 0)),
        out_shape=jax.ShapeDtypeStruct((NC, N, DH), jnp.float32),
    )(agg1, nrm, bg0, Wg1)


def _tc3_body(agg_ref, nrm_ref, bg1_ref, wc_ref, bc_ref, out_ref):
    h2 = jnp.maximum(
        agg_ref[...] * nrm_ref[:, 1][:, None] + bg1_ref[...], 0.0)
    out_ref[...] = _dot3(h2, wc_ref[...]) + bc_ref[...]


def _tc3(agg2, nrm, bg1, Wc, bc):
    full = lambda *shape: pl.BlockSpec(shape, lambda i: (0,) * len(shape))
    return pl.pallas_call(
        _tc3_body,
        grid=(N // BLK,),
        in_specs=[
            pl.BlockSpec((BLK, D), lambda i: (i, 0)),
            pl.BlockSpec((BLK, 8), lambda i: (i, 0)),
            full(D), full(D, NCLASS), full(NCLASS),
        ],
        out_specs=pl.BlockSpec((BLK, NCLASS), lambda i: (i, 0)),
        out_shape=jax.ShapeDtypeStruct((N, NCLASS), jnp.float32),
    )(agg2, nrm, bg1, Wc, bc)


def kernel(x, edge_index, W1, b1, W2, b2, Wg0, bg0, Wg1, bg1, Wc, bc):
    edge3 = edge_index.reshape(2, NCH, CHUNK)
    hist = _degrees(edge3)
    deg = hist[0] + hist[1]
    dsrc = deg[0, :, :ROWS_PT].reshape(N)
    ddst = deg[1, :, :ROWS_PT].reshape(N)
    o = jnp.where(dsrc > 0, lax.rsqrt(jnp.maximum(dsrc, 1.0)), 0.0)
    i = jnp.where(ddst > 0, lax.rsqrt(jnp.maximum(ddst, 1.0)), 0.0)
    nrm = jnp.stack([o, i, o, i, o, i, o, i], axis=1)
    h = _tc0(x, W1, b1, W2, b2)
    m1 = _tc1(h, Wg0, nrm)
    agg1 = _gconv(m1, edge3)
    m2 = _tc2(agg1, nrm, bg0, Wg1)
    agg2 = _gconv(m2, edge3)
    return _tc3(agg2, nrm, bg1, Wc, bc)


# speedup vs baseline: 1.0380x; 1.0380x over previous
import jax
import jax.numpy as jnp
from jax import lax
from jax.experimental import pallas as pl
from jax.experimental.pallas import tpu as pltpu
from jax.experimental.pallas import tpu_sc as plsc

N = 10000
E = 320000
D = 128
NCLASS = 64

NC = 2
NS = 16
DH = D // NC
CHUNK = 128
NCH = E // CHUNK
ROWS_PT = N // NS
HW = 16

GC_PT = NCH // NS
GC_EXTRA = NCH - GC_PT * NS
GC_NBUF = 6
DG_SC = NCH // NC
DG_PT = DG_SC // NS
DG_EXTRA = DG_SC - DG_PT * NS


def _sc_mesh():
    return plsc.VectorSubcoreMesh(
        core_axis_name="c", subcore_axis_name="s", num_cores=NC,
        num_subcores=NS,
    )


_SC_PARAMS = pltpu.CompilerParams(use_tc_tiling_on_sc=False)


def _deg_body(edge_hbm, hist_hbm, src_buf, dst_buf, obuf_s, obuf_d, zbuf,
              cb, hs, sem_s, sem_d):
    c = lax.axis_index("c")
    s = lax.axis_index("s")
    src_hbm = edge_hbm.at[0]
    dst_hbm = edge_hbm.at[1]
    base = c * DG_SC + s * DG_PT

    pltpu.sync_copy(src_hbm.at[pl.ds(base, DG_PT)],
                    src_buf.at[pl.ds(0, DG_PT)])
    pltpu.sync_copy(dst_hbm.at[pl.ds(base, DG_PT)],
                    dst_buf.at[pl.ds(0, DG_PT)])

    @pl.when(s < DG_EXTRA)
    def _():
        extra = c * DG_SC + NS * DG_PT + s
        pltpu.sync_copy(src_hbm.at[extra], src_buf.at[DG_PT])
        pltpu.sync_copy(dst_hbm.at[extra], dst_buf.at[DG_PT])

    lane = lax.iota(jnp.int32, 16)
    line_s = jnp.where(lane < 8, 1.0, 0.0).astype(jnp.float32)
    line_d = jnp.where(lane < 8, 0.0, 1.0).astype(jnp.float32)
    zline = jnp.zeros((16,), jnp.float32)

    def fill(i, _):
        obuf_s[i, :] = line_s
        obuf_d[i, :] = line_d
        return 0
    lax.fori_loop(0, CHUNK, fill, 0)

    def zfill(i, _):
        zbuf[i, :] = zline
        return 0
    lax.fori_loop(0, ROWS_PT, zfill, 0)

    pltpu.sync_copy(zbuf, hs.at[pl.ds(s * ROWS_PT, ROWS_PT)])
    plsc.subcore_barrier()

    nmine = DG_PT + jnp.where(
        s < DG_EXTRA, jnp.int32(1), jnp.int32(0))

    def step(j, _):
        pltpu.async_copy(obuf_s, hs.at[src_buf.at[j]], sem_s, add=True)
        pltpu.async_copy(obuf_d, hs.at[dst_buf.at[j]], sem_d, add=True)
        return 0
    lax.fori_loop(0, nmine, step, 0)

    def drain(j, _):
        pltpu.make_async_copy(obuf_s, hs.at[src_buf.at[0]], sem_s).wait()
        pltpu.make_async_copy(obuf_d, hs.at[dst_buf.at[0]], sem_d).wait()
        return 0
    lax.fori_loop(0, nmine, drain, 0)

    plsc.subcore_barrier()

    rows = pl.ds(s * ROWS_PT, ROWS_PT)
    pltpu.sync_copy(hs.at[rows], zbuf)
    for g in range(79):
        r0 = min(g * 8, ROWS_PT - 8)
        w = zbuf[r0, :]
        for k in range(1, 8):
            v = zbuf[r0 + k, :]
            w = jnp.where(lane == k, v, w)
            w = jnp.where(lane == 8 + k, v, w)
        cb[pl.ds(g * 16, 16)] = w
    pltpu.sync_copy(cb, hist_hbm.at[c, s])


def _degrees(edge3):
    f = pl.kernel(
        _deg_body,
        out_type=jax.ShapeDtypeStruct((NC, NS, 79 * 16), jnp.float32),
        mesh=_sc_mesh(),
        scratch_types=[
            pltpu.VMEM((DG_PT + 1, CHUNK), jnp.int32),
            pltpu.VMEM((DG_PT + 1, CHUNK), jnp.int32),
            pltpu.VMEM((CHUNK, HW), jnp.float32),
            pltpu.VMEM((CHUNK, HW), jnp.float32),
            pltpu.VMEM((ROWS_PT, HW), jnp.float32),
            pltpu.VMEM((79 * 16,), jnp.float32),
            pltpu.VMEM_SHARED((N, HW), jnp.float32),
            pltpu.SemaphoreType.DMA,
            pltpu.SemaphoreType.DMA,
        ],
        compiler_params=_SC_PARAMS,
    )
    return f(edge3)


def _gconv_body(m_hbm, edge_hbm, out_hbm, src_buf, dst_buf,
                rb0, rb1, rb2, rb3, rb4, rb5, agg_sh, gsem, ssem):
    c = lax.axis_index("c")
    s = lax.axis_index("s")
    src_hbm = edge_hbm.at[0]
    dst_hbm = edge_hbm.at[1]
    rb = (rb0, rb1, rb2, rb3, rb4, rb5)
    mh = m_hbm.at[c]

    pltpu.sync_copy(src_hbm.at[pl.ds(s * GC_PT, GC_PT)], src_buf)
    pltpu.sync_copy(dst_hbm.at[pl.ds(s * GC_PT, GC_PT)], dst_buf)

    zline = jnp.zeros((16,), jnp.float32)

    def zfill(i, _):
        for k in range(DH // 16):
            rb0[i, pl.ds(k * 16, 16)] = zline
        return 0
    lax.fori_loop(0, CHUNK, zfill, 0)

    for t in range(4):
        pltpu.sync_copy(
            rb0, agg_sh.at[pl.ds(s * ROWS_PT + t * CHUNK, CHUNK)])
    pltpu.sync_copy(rb0.at[pl.ds(0, ROWS_PT - 4 * CHUNK)],
                    agg_sh.at[pl.ds(s * ROWS_PT + 4 * CHUNK,
                                    ROWS_PT - 4 * CHUNK)])
    plsc.subcore_barrier()

    def start_gather(j, b):
        pltpu.async_copy(mh.at[src_buf.at[j]], rb[b], gsem.at[b])

    def wait_gather(b):
        pltpu.make_async_copy(mh.at[src_buf.at[0]], rb[b], gsem.at[b]).wait()

    def start_scatter(j, b):
        pltpu.async_copy(rb[b], agg_sh.at[dst_buf.at[j]], ssem.at[b],
                         add=True)

    def wait_scatter(b):
        pltpu.make_async_copy(
            rb[b], agg_sh.at[dst_buf.at[0]], ssem.at[b]).wait()

    for b in range(GC_NBUF):
        start_gather(b, b)

    def step(g, _):
        j0 = g * GC_NBUF
        for b in range(GC_NBUF):
            wait_gather(b)
            start_scatter(j0 + b, b)
        for b in range(GC_NBUF):
            wait_scatter(b)
            start_gather(j0 + GC_NBUF + b, b)
        return 0
    lax.fori_loop(0, GC_PT // GC_NBUF - 1, step, 0)

    j0 = GC_PT - GC_NBUF
    for b in range(GC_NBUF):
        wait_gather(b)
        start_scatter(j0 + b, b)
    for b in range(GC_NBUF):
        wait_scatter(b)

    @pl.when(s < GC_EXTRA)
    def _():
        pltpu.sync_copy(src_hbm.at[NS * GC_PT + s], src_buf.at[0])
        pltpu.sync_copy(dst_hbm.at[NS * GC_PT + s], dst_buf.at[0])
        pltpu.async_copy(mh.at[src_buf.at[0]], rb[0], gsem.at[0])
        wait_gather(0)
        pltpu.sync_copy(rb[0], agg_sh.at[dst_buf.at[0]], add=True)

    plsc.subcore_barrier()
    rows = pl.ds(s * ROWS_PT, ROWS_PT)
    pltpu.sync_copy(agg_sh.at[rows], out_hbm.at[rows, pl.ds(c * DH, DH)])


def _gconv(m, edge3):
    f = pl.kernel(
        _gconv_body,
        out_type=jax.ShapeDtypeStruct((N, D), jnp.float32),
        mesh=_sc_mesh(),
        scratch_types=[
            pltpu.VMEM((GC_PT, CHUNK), jnp.int32),
            pltpu.VMEM((GC_PT, CHUNK), jnp.int32),
        ] + [pltpu.VMEM((CHUNK, DH), jnp.float32) for _ in range(GC_NBUF)] + [
            pltpu.VMEM_SHARED((N, DH), jnp.float32),
            pltpu.SemaphoreType.DMA((GC_NBUF,)),
            pltpu.SemaphoreType.DMA((GC_NBUF,)),
        ],
        compiler_params=_SC_PARAMS,
    )
    return f(m, edge3)


BLK = 1000


def _dot3(a, b):
    ah = a.astype(jnp.bfloat16)
    al = (a - ah.astype(jnp.float32)).astype(jnp.bfloat16)
    bh = b.astype(jnp.bfloat16)
    bl = (b - bh.astype(jnp.float32)).astype(jnp.bfloat16)
    f = lambda p, q: jax.lax.dot_general(
        p, q, (((1,), (0,)), ((), ())),
        preferred_element_type=jnp.float32)
    return f(ah, bh) + f(ah, bl) + f(al, bh)


def _tc0_body(x_ref, w1_ref, b1_ref, w2_ref, b2_ref, h_ref):
    h = _dot3(x_ref[...], w1_ref[...]) + b1_ref[...]
    h_ref[...] = _dot3(h, w2_ref[...]) + b2_ref[...]


def _tc0(x, W1, b1, W2, b2):
    full = lambda *shape: pl.BlockSpec(shape, lambda i: (0,) * len(shape))
    return pl.pallas_call(
        _tc0_body,
        grid=(N // BLK,),
        in_specs=[
            pl.BlockSpec((BLK, D), lambda i: (i, 0)),
            full(D, D), full(D), full(D, D), full(D),
        ],
        out_specs=pl.BlockSpec((BLK, D), lambda i: (i, 0)),
        out_shape=jax.ShapeDtypeStruct((N, D), jnp.float32),
    )(x, W1, b1, W2, b2)


def _tc1_body(h_ref, wg0_ref, nrm_ref, m1_ref):
    z = _dot3(h_ref[...], wg0_ref[...])
    m = z * nrm_ref[:, 0][:, None]
    m1_ref[0] = m[:, :DH]
    m1_ref[1] = m[:, DH:]


def _tc1(h, Wg0, nrm):
    full = lambda *shape: pl.BlockSpec(shape, lambda i: (0,) * len(shape))
    return pl.pallas_call(
        _tc1_body,
        grid=(N // BLK,),
        in_specs=[
            pl.BlockSpec((BLK, D), lambda i: (i, 0)),
            full(D, D),
            pl.BlockSpec((BLK, 8), lambda i: (i, 0)),
        ],
        out_specs=pl.BlockSpec((NC, BLK, DH), lambda i: (0, i, 0)),
        out_shape=jax.ShapeDtypeStruct((NC, N, DH), jnp.float32),
    )(h, Wg0, nrm)


def _tc2_body(agg_ref, nrm_ref, bg0_ref, wg1_ref, m2_ref):
    h1 = jnp.maximum(
        agg_ref[...] * nrm_ref[:, 1][:, None] + bg0_ref[...], 0.0)
    m = _dot3(h1, wg1_ref[...]) * nrm_ref[:, 0][:, None]
    m2_ref[0] = m[:, :DH]
    m2_ref[1] = m[:, DH:]


def _tc2(agg1, nrm, bg0, Wg1):
    full = lambda *shape: pl.BlockSpec(shape, lambda i: (0,) * len(shape))
    return pl.pallas_call(
        _tc2_body,
        grid=(N // BLK,),
        in_specs=[
            pl.BlockSpec((BLK, D), lambda i: (i, 0)),
            pl.BlockSpec((BLK, 8), lambda i: (i, 0)),
            full(D), full(D, D),
        ],
        out_specs=pl.BlockSpec((NC, BLK, DH), lambda i: (0, i, 0)),
        out_shape=jax.ShapeDtypeStruct((NC, N, DH), jnp.float32),
    )(agg1, nrm, bg0, Wg1)


def _tc3_body(agg_ref, nrm_ref, bg1_ref, wc_ref, bc_ref, out_ref):
    h2 = jnp.maximum(
        agg_ref[...] * nrm_ref[:, 1][:, None] + bg1_ref[...], 0.0)
    out_ref[...] = _dot3(h2, wc_ref[...]) + bc_ref[...]


def _tc3(agg2, nrm, bg1, Wc, bc):
    full = lambda *shape: pl.BlockSpec(shape, lambda i: (0,) * len(shape))
    return pl.pallas_call(
        _tc3_body,
        grid=(N // BLK,),
        in_specs=[
            pl.BlockSpec((BLK, D), lambda i: (i, 0)),
            pl.BlockSpec((BLK, 8), lambda i: (i, 0)),
            full(D), full(D, NCLASS), full(NCLASS),
        ],
        out_specs=pl.BlockSpec((BLK, NCLASS), lambda i: (i, 0)),
        out_shape=jax.ShapeDtypeStruct((N, NCLASS), jnp.float32),
    )(agg2, nrm, bg1, Wc, bc)


def kernel(x, edge_index, W1, b1, W2, b2, Wg0, bg0, Wg1, bg1, Wc, bc):
    edge3 = edge_index.reshape(2, NCH, CHUNK)
    hist = _degrees(edge3).reshape(NC, NS, 79, 2, 8)
    deg = hist[0] + hist[1]
    flat = deg.transpose(2, 0, 1, 3).reshape(2, NS, 632)
    dsrc = jnp.concatenate(
        [flat[0, :, :624], flat[0, :, 631:632]], axis=1).reshape(N)
    ddst = jnp.concatenate(
        [flat[1, :, :624], flat[1, :, 631:632]], axis=1).reshape(N)
    o = jnp.where(dsrc > 0, lax.rsqrt(jnp.maximum(dsrc, 1.0)), 0.0)
    i = jnp.where(ddst > 0, lax.rsqrt(jnp.maximum(ddst, 1.0)), 0.0)
    nrm = jnp.stack([o, i, o, i, o, i, o, i], axis=1)
    h = _tc0(x, W1, b1, W2, b2)
    m1 = _tc1(h, Wg0, nrm)
    agg1 = _gconv(m1, edge3)
    m2 = _tc2(agg1, nrm, bg0, Wg1)
    agg2 = _gconv(m2, edge3)
    return _tc3(agg2, nrm, bg1, Wc, bc)
